# baseline (speedup 1.0000x reference)
import functools
import jax
import jax.numpy as jnp
from jax import lax
from jax.experimental import pallas as pl
from jax.experimental.pallas import tpu as pltpu
from jax.experimental.pallas import tpu_sc as plsc

BATCH = 4096
HIST = 200
D = 64
TOTAL = BATCH * HIST

_info = plsc.get_sparse_core_info()
NC, NS, NL = _info.num_cores, _info.num_subcores, _info.num_lanes
NW = NC * NS
B_PER_W = TOTAL // NW
CHUNK = 128
N_CHUNKS = B_PER_W // CHUNK

_mesh = plsc.VectorSubcoreMesh(core_axis_name="c", subcore_axis_name="s")

VOCAB = 1000000
VB = 128
NBLK = VOCAB // VB
TAIL0 = NBLK * VB
KMAX = -(-NBLK // NW)


@functools.partial(
    pl.kernel,
    mesh=_mesh,
    out_type=jax.ShapeDtypeStruct((VOCAB, 2 * D), jnp.float32),
    scratch_types=[
        pltpu.VMEM((2, D, VB), jnp.float32),
        pltpu.VMEM((2, VB, 2 * D), jnp.float32),
        pltpu.SemaphoreType.DMA,
        pltpu.SemaphoreType.DMA,
    ],
    compiler_params=pltpu.CompilerParams(needs_layout_passes=False),
)
def _repack_kernel(tab_t_hbm, tail_hbm, out_hbm, tin, tout, isem, osem):
    wid = lax.axis_index("s") * NC + lax.axis_index("c")

    def col0(k):
        return (wid + NW * k) * VB

    def in_desc(k, b):
        return pltpu.make_async_copy(
            tab_t_hbm.at[:, pl.ds(col0(k), VB)], tin.at[b], isem)

    def out_desc(k, b):
        return pltpu.make_async_copy(
            tout.at[b], out_hbm.at[pl.ds(col0(k), VB)], osem)

    def transpose(b):
        for c0 in range(0, VB, NL):
            rowv = lax.iota(jnp.int32, NL) + c0
            for j in range(D):
                x = tin[b, j, pl.ds(c0, NL)]
                jv = lax.iota(jnp.int32, NL) * 0 + j
                plsc.store_scatter(tout.at[b], [rowv, jv], x)

    def valid(k):
        return wid + NW * k < NBLK

    @pl.when(valid(0))
    def _():
        in_desc(0, 0).start()

    def body(g, carry):
        k0 = 2 * g
        k1 = k0 + 1

        @pl.when(valid(k0))
        def _():
            @pl.when(g > 0)
            def _():
                out_desc(k0 - 2, 0).wait()

            @pl.when(valid(k1))
            def _():
                in_desc(k1, 1).start()

            in_desc(k0, 0).wait()
            transpose(0)
            out_desc(k0, 0).start()

        @pl.when(valid(k1))
        def _():
            @pl.when(g > 0)
            def _():
                out_desc(k1 - 2, 1).wait()

            @pl.when(valid(k1 + 1))
            def _():
                in_desc(k1 + 1, 0).start()

            in_desc(k1, 1).wait()
            transpose(1)
            out_desc(k1, 1).start()

        return carry

    lax.fori_loop(0, (KMAX + 1) // 2, body, 0)
    last_e = 2 * ((KMAX - 1) // 2)
    last_o = last_e + 1 if last_e + 1 < KMAX else last_e - 1

    @pl.when(valid(last_e))
    def _():
        out_desc(last_e, 0).wait()

    @pl.when(jnp.logical_and(jnp.logical_not(valid(last_e)),
                             valid(last_e - 2)))
    def _():
        out_desc(last_e - 2, 0).wait()

    @pl.when(valid(last_o))
    def _():
        out_desc(last_o, 1).wait()

    @pl.when(jnp.logical_and(jnp.logical_not(valid(last_o)),
                             valid(last_o - 2)))
    def _():
        out_desc(last_o - 2, 1).wait()

    @pl.when(wid == NBLK % NW)
    def _():
        pltpu.sync_copy(tail_hbm, out_hbm.at[pl.ds(TAIL0, VOCAB - TAIL0)])


@functools.partial(
    pl.kernel,
    mesh=_mesh,
    out_type=jax.ShapeDtypeStruct((TOTAL, D), jnp.float32),
    scratch_types=[
        pltpu.VMEM((B_PER_W,), jnp.int32),
        pltpu.VMEM((2, CHUNK, 2 * D), jnp.float32),
        pltpu.VMEM((2, CHUNK, D), jnp.float32),
        pltpu.VMEM((CHUNK,), jnp.int32),
        pltpu.VMEM((CHUNK,), jnp.int32),
        pltpu.SemaphoreType.DMA,
        pltpu.SemaphoreType.DMA,
    ],
)
def _gather_kernel(table_hbm, idx_hbm, out_hbm, idx_v, pairs_v, rows_v,
                   rowidx0_v, rowidx1_v, gsem, wsem):
    rowidx_bufs = (rowidx0_v, rowidx1_v)
    wid = lax.axis_index("s") * NC + lax.axis_index("c")
    base = wid * B_PER_W
    pltpu.sync_copy(idx_hbm.at[pl.ds(base, B_PER_W)], idx_v)

    def prep(i, b):
        def grp(g, carry):
            rowidx_bufs[b][pl.ds(g * NL, NL)] = (
                idx_v[pl.ds(i * CHUNK + g * NL, NL)])
            return carry

        lax.fori_loop(0, CHUNK // NL, grp, 0, unroll=4)

    def gather_desc(i, b):
        return pltpu.make_async_copy(
            table_hbm.at[rowidx_bufs[b]], pairs_v.at[b], gsem)

    def compact(b):
        def row(r, carry):
            for j0 in range(0, D, NL):
                rows_v[b, r, pl.ds(j0, NL)] = pairs_v[b, r, pl.ds(j0, NL)]
            return carry

        lax.fori_loop(0, CHUNK, row, 0)

    def write_desc(i, b):
        return pltpu.make_async_copy(
            rows_v.at[b], out_hbm.at[pl.ds(base + i * CHUNK, CHUNK)], wsem)

    prep(0, 0)
    gather_desc(0, 0).start()

    def body(g, carry):
        i0 = 2 * g
        i1 = i0 + 1

        @pl.when(g > 0)
        def _():
            write_desc(i0 - 2, 0).wait()

        prep(i1, 1)
        gather_desc(i1, 1).start()
        gather_desc(i0, 0).wait()
        compact(0)
        write_desc(i0, 0).start()

        @pl.when(g > 0)
        def _():
            write_desc(i1 - 2, 1).wait()

        @pl.when(i0 + 2 < N_CHUNKS)
        def _():
            prep(i0 + 2, 0)
            gather_desc(i0 + 2, 0).start()

        gather_desc(i1, 1).wait()
        compact(1)
        write_desc(i1, 1).start()
        return carry

    lax.fori_loop(0, N_CHUNKS // 2, body, 0)
    write_desc(N_CHUNKS - 2, 0).wait()
    write_desc(N_CHUNKS - 1, 1).wait()


def kernel(indices, table):
    idx_flat = indices.reshape(TOTAL).astype(jnp.int32)
    tail = jnp.pad(table[TAIL0:, :], ((0, 0), (0, D)))
    table_pad = _repack_kernel(jnp.swapaxes(table, 0, 1), tail)
    out = _gather_kernel(table_pad, idx_flat)
    return out.reshape(BATCH, HIST, D)


# speedup vs baseline: 1.6772x; 1.6772x over previous
import functools
import jax
import jax.numpy as jnp
from jax import lax
from jax.experimental import pallas as pl
from jax.experimental.pallas import tpu as pltpu
from jax.experimental.pallas import tpu_sc as plsc

BATCH = 4096
HIST = 200
D = 64
TOTAL = BATCH * HIST

_info = plsc.get_sparse_core_info()
NC, NS, NL = _info.num_cores, _info.num_subcores, _info.num_lanes
NW = NC * NS
B_PER_W = TOTAL // NW
CHUNK = 200
N_CHUNKS = B_PER_W // CHUNK

_mesh = plsc.VectorSubcoreMesh(core_axis_name="c", subcore_axis_name="s")


@functools.partial(
    pl.kernel,
    mesh=_mesh,
    out_type=jax.ShapeDtypeStruct((TOTAL, D), jnp.float32),
    scratch_types=[
        pltpu.VMEM((2, CHUNK, 2 * D), jnp.float32),
        pltpu.VMEM((2, CHUNK, D), jnp.float32),
        pltpu.VMEM((CHUNK,), jnp.int32),
        pltpu.VMEM((CHUNK,), jnp.int32),
        pltpu.SemaphoreType.DMA,
        pltpu.SemaphoreType.DMA,
    ],
)
def _gather_kernel(table_hbm, idx_hbm, out_hbm, pairs_v, rows_v,
                   rowidx0_v, rowidx1_v, gsem, wsem):
    rowidx_bufs = (rowidx0_v, rowidx1_v)
    wid = lax.axis_index("s") * NC + lax.axis_index("c")
    base = wid * B_PER_W

    def stage_idx(i, b):
        pltpu.sync_copy(
            idx_hbm.at[pl.ds(base + i * CHUNK, CHUNK)], rowidx_bufs[b])

    def gather_desc(i, b):
        return pltpu.make_async_copy(
            table_hbm.at[rowidx_bufs[b]], pairs_v.at[b], gsem)

    def compact(b):
        def row(r, carry):
            for j0 in range(0, D, NL):
                rows_v[b, r, pl.ds(j0, NL)] = pairs_v[b, r, pl.ds(j0, NL)]
            return carry

        lax.fori_loop(0, CHUNK, row, 0)

    def write_desc(i, b):
        return pltpu.make_async_copy(
            rows_v.at[b], out_hbm.at[pl.ds(base + i * CHUNK, CHUNK)], wsem)

    stage_idx(0, 0)
    gather_desc(0, 0).start()

    def body(g, carry):
        i0 = 2 * g
        i1 = i0 + 1

        @pl.when(g > 0)
        def _():
            write_desc(i0 - 2, 0).wait()

        stage_idx(i1, 1)
        gather_desc(i1, 1).start()
        gather_desc(i0, 0).wait()
        compact(0)
        write_desc(i0, 0).start()

        @pl.when(g > 0)
        def _():
            write_desc(i1 - 2, 1).wait()

        @pl.when(i0 + 2 < N_CHUNKS)
        def _():
            stage_idx(i0 + 2, 0)
            gather_desc(i0 + 2, 0).start()

        gather_desc(i1, 1).wait()
        compact(1)
        write_desc(i1, 1).start()
        return carry

    lax.fori_loop(0, N_CHUNKS // 2, body, 0)
    write_desc(N_CHUNKS - 2, 0).wait()
    write_desc(N_CHUNKS - 1, 1).wait()


def kernel(indices, table):
    idx_flat = indices.reshape(TOTAL).astype(jnp.int32)
    table_pad = jnp.pad(table, ((0, 0), (0, D)))
    out = _gather_kernel(table_pad, idx_flat)
    return out.reshape(BATCH, HIST, D)
